# Initial kernel scaffold; baseline (speedup 1.0000x reference)
import jax
import jax.numpy as jnp
from jax.experimental import pallas as pl


def kernel(x, edge_index, edge_weights, W_f, b_f, W_b, b_b):
    raise NotImplementedError("write your pallas kernel here")


# speedup vs baseline: 7.7127x; 7.7127x over previous
import functools

import jax
import jax.numpy as jnp
from jax import lax
from jax.experimental import pallas as pl
from jax.experimental.pallas import tpu as pltpu
from jax.experimental.pallas import tpu_sc as plsc

N = 10000
D = 128
E = 320000
W = 144
NC = 2
NS = 16
NW = NC * NS
K = 128
CPT = 80
EP = NW * K * CPT
PAD = EP - E
NACC = 10240
RPT = NACC // NS


def _sc_body(xaug, src2, dst2, zeros, out, acc, rows, src_t, dst_t, sem):
    c = lax.axis_index("c")
    s = lax.axis_index("s")
    wid = s * NC + c

    pltpu.sync_copy(zeros.at[pl.ds(s * RPT, RPT)], acc.at[pl.ds(s * RPT, RPT)])

    base = wid * CPT
    pltpu.sync_copy(src2.at[pl.ds(base, CPT)], src_t)
    pltpu.sync_copy(dst2.at[pl.ds(base, CPT)], dst_t)
    plsc.subcore_barrier()

    def chunk(j, carry):
        pltpu.async_copy(xaug.at[src_t.at[j]], rows, sem).wait()
        pltpu.sync_copy(rows, acc.at[dst_t.at[j]], add=True)
        return carry

    lax.fori_loop(0, CPT, chunk, 0)
    plsc.subcore_barrier()

    pltpu.sync_copy(acc.at[pl.ds(s * RPT, RPT)], out.at[c].at[pl.ds(s * RPT, RPT)])


@jax.jit
def _sc_accumulate(xaug, src2, dst2, zeros):
    mesh = plsc.VectorSubcoreMesh(core_axis_name="c", subcore_axis_name="s")
    return pl.kernel(
        _sc_body,
        out_type=jax.ShapeDtypeStruct((NC, NACC, W), jnp.float32),
        mesh=mesh,
        scratch_types=[
            pltpu.VMEM_SHARED((NACC, W), jnp.float32),
            pltpu.VMEM((K, W), jnp.float32),
            pltpu.VMEM((CPT, K), jnp.int32),
            pltpu.VMEM((CPT, K), jnp.int32),
            pltpu.SemaphoreType.DMA,
        ],
        compiler_params=pltpu.CompilerParams(use_tc_tiling_on_sc=False),
    )(xaug, src2, dst2, zeros)


def _tc_body(p_ref, o_ref):
    p = p_ref[...]
    ssum = p[0] + p[1]
    lane = lax.broadcasted_iota(jnp.int32, ssum.shape, 1)
    is_feat = lane < D
    deg16 = jnp.where(is_feat, 0.0, ssum)
    deg = jnp.sum(deg16, axis=1, keepdims=True) * (1.0 / 16.0)
    aggr = ssum / jnp.maximum(deg, 1.0)
    feat = jnp.where(is_feat, aggr, 0.0)
    nrm = jnp.sqrt(jnp.sum(feat * feat, axis=1, keepdims=True))
    o = jnp.maximum(2.0 * feat / jnp.maximum(nrm, 1e-12), 0.0)
    o_ref[...] = o[:, :D]


@jax.jit
def _tc_normalize(parts):
    R = 1000
    return pl.pallas_call(
        _tc_body,
        grid=(N // R,),
        in_specs=[pl.BlockSpec((NC, R, W), lambda i: (0, i, 0))],
        out_specs=pl.BlockSpec((R, D), lambda i: (i, 0)),
        out_shape=jax.ShapeDtypeStruct((N, D), jnp.float32),
    )(parts)


def kernel(x, edge_index, edge_weights, W_f, b_f, W_b, b_b):
    src = edge_index[0]
    dst = edge_index[1]
    xaug = jnp.concatenate([x, jnp.ones((N, 16), jnp.float32)], axis=1)
    ar = jnp.arange(PAD, dtype=jnp.int32)
    src_p = jnp.concatenate([src, ar % N])
    dst_p = jnp.concatenate([dst, N + (ar % 16)])
    src2 = src_p.reshape(EP // K, K)
    dst2 = dst_p.reshape(EP // K, K)
    zeros = jnp.zeros((NACC, W), jnp.float32)
    parts = _sc_accumulate(xaug, src2, dst2, zeros)
    return _tc_normalize(parts)


# speedup vs baseline: 9.6239x; 1.2478x over previous
import functools

import jax
import jax.numpy as jnp
from jax import lax
from jax.experimental import pallas as pl
from jax.experimental.pallas import tpu as pltpu
from jax.experimental.pallas import tpu_sc as plsc

N = 10000
D = 128
E = 320000
W = 144
NC = 2
NS = 16
NW = NC * NS
K = 32
CPT = 320
EP = NW * K * CPT
PAD = EP - E
NACC = 10112
RPT = NACC // NS


NBUF = 4


def _sc_body(xaug, src2, dst2, zeros, out, acc,
             rows0, rows1, rows2, rows3, src_t, dst_t,
             sg0, sg1, sg2, sg3, ss0, ss1, ss2, ss3):
    c = lax.axis_index("c")
    s = lax.axis_index("s")
    wid = s * NC + c
    rows = (rows0, rows1, rows2, rows3)
    sg = (sg0, sg1, sg2, sg3)
    ss = (ss0, ss1, ss2, ss3)

    pltpu.sync_copy(zeros.at[pl.ds(s * RPT, RPT)], acc.at[pl.ds(s * RPT, RPT)])

    base = wid * CPT
    pltpu.sync_copy(src2.at[pl.ds(base, CPT)], src_t)
    pltpu.sync_copy(dst2.at[pl.ds(base, CPT)], dst_t)
    plsc.subcore_barrier()

    for b in range(NBUF):
        pltpu.async_copy(xaug.at[src_t.at[b]], rows[b], sg[b])

    def wait_gather(b):
        pltpu.make_async_copy(xaug.at[src_t.at[0]], rows[b], sg[b]).wait()

    def step(j4, carry):
        j = j4 * NBUF
        scatters = []
        for b in range(NBUF):
            wait_gather(b)
            scatters.append(
                pltpu.async_copy(rows[b], acc.at[dst_t.at[j + b]], ss[b], add=True))
        for b in range(NBUF):
            scatters[b].wait()
            jn = jnp.minimum(j + b + NBUF, CPT - 1)
            pltpu.async_copy(xaug.at[src_t.at[jn]], rows[b], sg[b])
        return carry

    lax.fori_loop(0, CPT // NBUF, step, 0)
    for b in range(NBUF):
        wait_gather(b)
    plsc.subcore_barrier()

    pltpu.sync_copy(acc.at[pl.ds(s * RPT, RPT)], out.at[c].at[pl.ds(s * RPT, RPT)])


@jax.jit
def _sc_accumulate(xaug, src2, dst2, zeros):
    mesh = plsc.VectorSubcoreMesh(core_axis_name="c", subcore_axis_name="s")
    return pl.kernel(
        _sc_body,
        out_type=jax.ShapeDtypeStruct((NC, NACC, W), jnp.float32),
        mesh=mesh,
        scratch_types=(
            [pltpu.VMEM_SHARED((NACC, W), jnp.float32)]
            + [pltpu.VMEM((K, W), jnp.float32) for _ in range(NBUF)]
            + [pltpu.VMEM((CPT, K), jnp.int32) for _ in range(2)]
            + [pltpu.SemaphoreType.DMA for _ in range(2 * NBUF)]
        ),
        compiler_params=pltpu.CompilerParams(use_tc_tiling_on_sc=False),
    )(xaug, src2, dst2, zeros)


def _tc_body(p_ref, o_ref):
    p = p_ref[...]
    ssum = p[0] + p[1]
    lane = lax.broadcasted_iota(jnp.int32, ssum.shape, 1)
    is_feat = lane < D
    deg16 = jnp.where(is_feat, 0.0, ssum)
    deg = jnp.sum(deg16, axis=1, keepdims=True) * (1.0 / 16.0)
    aggr = ssum / jnp.maximum(deg, 1.0)
    feat = jnp.where(is_feat, aggr, 0.0)
    nrm = jnp.sqrt(jnp.sum(feat * feat, axis=1, keepdims=True))
    o = jnp.maximum(2.0 * feat / jnp.maximum(nrm, 1e-12), 0.0)
    o_ref[...] = o[:, :D]


@jax.jit
def _tc_normalize(parts):
    R = 1000
    return pl.pallas_call(
        _tc_body,
        grid=(N // R,),
        in_specs=[pl.BlockSpec((NC, R, W), lambda i: (0, i, 0))],
        out_specs=pl.BlockSpec((R, D), lambda i: (i, 0)),
        out_shape=jax.ShapeDtypeStruct((N, D), jnp.float32),
    )(parts)


def kernel(x, edge_index, edge_weights, W_f, b_f, W_b, b_b):
    src = edge_index[0]
    dst = edge_index[1]
    xaug = jnp.concatenate([x, jnp.ones((N, 16), jnp.float32)], axis=1)
    ar = jnp.arange(PAD, dtype=jnp.int32)
    src_p = jnp.concatenate([src, ar % N])
    dst_p = jnp.concatenate([dst, N + (ar % 16)])
    src2 = src_p.reshape(EP // K, K)
    dst2 = dst_p.reshape(EP // K, K)
    zeros = jnp.zeros((NACC, W), jnp.float32)
    parts = _sc_accumulate(xaug, src2, dst2, zeros)
    return _tc_normalize(parts)


# speedup vs baseline: 12.2503x; 1.2729x over previous
import jax
import jax.numpy as jnp
from jax import lax
from jax.experimental import pallas as pl
from jax.experimental.pallas import tpu as pltpu
from jax.experimental.pallas import tpu_sc as plsc

N = 10000
D = 128
E = 320000
NC = 2
NS = 16
NW = NC * NS
K = 32
CPT = 320
EP = NW * K * CPT
PAD = EP - E
NACC = 10112
RPT = NACC // NS
NBUF = 4


def _sc_body(x, src2, dst2, zeros, out, acc,
             rows0, rows1, rows2, rows3, src_t, dst_t,
             sg0, sg1, sg2, sg3, ss0, ss1, ss2, ss3):
    c = lax.axis_index("c")
    s = lax.axis_index("s")
    wid = s * NC + c
    rows = (rows0, rows1, rows2, rows3)
    sg = (sg0, sg1, sg2, sg3)
    ss = (ss0, ss1, ss2, ss3)

    pltpu.sync_copy(zeros.at[pl.ds(s * RPT, RPT)], acc.at[pl.ds(s * RPT, RPT)])

    base = wid * CPT
    pltpu.sync_copy(src2.at[pl.ds(base, CPT)], src_t)
    pltpu.sync_copy(dst2.at[pl.ds(base, CPT)], dst_t)
    plsc.subcore_barrier()

    for b in range(NBUF):
        pltpu.async_copy(x.at[src_t.at[b]], rows[b], sg[b])

    def wait_gather(b):
        pltpu.make_async_copy(x.at[src_t.at[0]], rows[b], sg[b]).wait()

    def step(j4, carry):
        j = j4 * NBUF
        scatters = []
        for b in range(NBUF):
            wait_gather(b)
            scatters.append(
                pltpu.async_copy(rows[b], acc.at[dst_t.at[j + b]], ss[b], add=True))
        for b in range(NBUF):
            scatters[b].wait()
            jn = jnp.minimum(j + b + NBUF, CPT - 1)
            pltpu.async_copy(x.at[src_t.at[jn]], rows[b], sg[b])
        return carry

    lax.fori_loop(0, CPT // NBUF, step, 0)
    for b in range(NBUF):
        wait_gather(b)
    plsc.subcore_barrier()

    pltpu.sync_copy(acc.at[pl.ds(s * RPT, RPT)], out.at[c].at[pl.ds(s * RPT, RPT)])


@jax.jit
def _sc_accumulate(x, src2, dst2, zeros):
    mesh = plsc.VectorSubcoreMesh(core_axis_name="c", subcore_axis_name="s")
    return pl.kernel(
        _sc_body,
        out_type=jax.ShapeDtypeStruct((NC, NACC, D), jnp.float32),
        mesh=mesh,
        scratch_types=(
            [pltpu.VMEM_SHARED((NACC, D), jnp.float32)]
            + [pltpu.VMEM((K, D), jnp.float32) for _ in range(NBUF)]
            + [pltpu.VMEM((CPT, K), jnp.int32) for _ in range(2)]
            + [pltpu.SemaphoreType.DMA for _ in range(2 * NBUF)]
        ),
        compiler_params=pltpu.CompilerParams(use_tc_tiling_on_sc=False),
    )(x, src2, dst2, zeros)


def _tc_body(p_ref, o_ref):
    p = p_ref[...]
    ssum = p[0] + p[1]
    nrm = jnp.sqrt(jnp.sum(ssum * ssum, axis=1, keepdims=True))
    o_ref[...] = jnp.maximum(2.0 * ssum / jnp.maximum(nrm, 1e-12), 0.0)


@jax.jit
def _tc_normalize(parts):
    R = 1000
    return pl.pallas_call(
        _tc_body,
        grid=(N // R,),
        in_specs=[pl.BlockSpec((NC, R, D), lambda i: (0, i, 0))],
        out_specs=pl.BlockSpec((R, D), lambda i: (i, 0)),
        out_shape=jax.ShapeDtypeStruct((N, D), jnp.float32),
    )(parts)


def kernel(x, edge_index, edge_weights, W_f, b_f, W_b, b_b):
    src = edge_index[0]
    dst = edge_index[1]
    ar = jnp.arange(PAD, dtype=jnp.int32)
    src_p = jnp.concatenate([src, ar % N])
    dst_p = jnp.concatenate([dst, N + (ar % 16)])
    src2 = src_p.reshape(EP // K, K)
    dst2 = dst_p.reshape(EP // K, K)
    zeros = jnp.zeros((NACC, D), jnp.float32)
    parts = _sc_accumulate(x, src2, dst2, zeros)
    return _tc_normalize(parts)


# speedup vs baseline: 12.9061x; 1.0535x over previous
import jax
import jax.numpy as jnp
from jax import lax
from jax.experimental import pallas as pl
from jax.experimental.pallas import tpu as pltpu
from jax.experimental.pallas import tpu_sc as plsc

N = 10000
D = 128
E = 320000
NC = 2
NS = 16
NW = NC * NS
K = 32
CPT = 320
EP = NW * K * CPT
PAD = EP - E
NACC = 10112
RPT = NACC // NS
NBUF = 5


def _sc_body(x, src2, dst2, zeros, out, acc,
             rows0, rows1, rows2, rows3, rows4, src_t, dst_t,
             sg0, sg1, sg2, sg3, sg4, ss0, ss1, ss2, ss3, ss4):
    c = lax.axis_index("c")
    s = lax.axis_index("s")
    wid = s * NC + c
    rows = (rows0, rows1, rows2, rows3, rows4)
    sg = (sg0, sg1, sg2, sg3, sg4)
    ss = (ss0, ss1, ss2, ss3, ss4)

    pltpu.sync_copy(zeros.at[pl.ds(s * RPT, RPT)], acc.at[pl.ds(s * RPT, RPT)])

    base = wid * CPT
    pltpu.sync_copy(src2.at[pl.ds(base, CPT)], src_t)
    pltpu.sync_copy(dst2.at[pl.ds(base, CPT)], dst_t)
    plsc.subcore_barrier()

    for b in range(NBUF):
        pltpu.async_copy(x.at[src_t.at[b]], rows[b], sg[b])

    def wait_gather(b):
        pltpu.make_async_copy(x.at[src_t.at[0]], rows[b], sg[b]).wait()

    def step(j4, carry):
        j = j4 * NBUF
        scatters = []
        for b in range(NBUF):
            wait_gather(b)
            scatters.append(
                pltpu.async_copy(rows[b], acc.at[dst_t.at[j + b]], ss[b], add=True))
        for b in range(NBUF):
            scatters[b].wait()
            jn = jnp.minimum(j + b + NBUF, CPT - 1)
            pltpu.async_copy(x.at[src_t.at[jn]], rows[b], sg[b])
        return carry

    lax.fori_loop(0, CPT // NBUF, step, 0)
    for b in range(NBUF):
        wait_gather(b)
    plsc.subcore_barrier()

    pltpu.sync_copy(acc.at[pl.ds(s * RPT, RPT)], out.at[c].at[pl.ds(s * RPT, RPT)])


@jax.jit
def _sc_accumulate(x, src2, dst2, zeros):
    mesh = plsc.VectorSubcoreMesh(core_axis_name="c", subcore_axis_name="s")
    return pl.kernel(
        _sc_body,
        out_type=jax.ShapeDtypeStruct((NC, NACC, D), jnp.float32),
        mesh=mesh,
        scratch_types=(
            [pltpu.VMEM_SHARED((NACC, D), jnp.float32)]
            + [pltpu.VMEM((K, D), jnp.float32) for _ in range(NBUF)]
            + [pltpu.VMEM((CPT, K), jnp.int32) for _ in range(2)]
            + [pltpu.SemaphoreType.DMA for _ in range(2 * NBUF)]
        ),
        compiler_params=pltpu.CompilerParams(use_tc_tiling_on_sc=False),
    )(x, src2, dst2, zeros)


def _tc_body(p_ref, o_ref):
    p = p_ref[...]
    ssum = p[0] + p[1]
    nrm = jnp.sqrt(jnp.sum(ssum * ssum, axis=1, keepdims=True))
    o_ref[...] = jnp.maximum(2.0 * ssum / jnp.maximum(nrm, 1e-12), 0.0)


@jax.jit
def _tc_normalize(parts):
    R = 1000
    return pl.pallas_call(
        _tc_body,
        grid=(N // R,),
        in_specs=[pl.BlockSpec((NC, R, D), lambda i: (0, i, 0))],
        out_specs=pl.BlockSpec((R, D), lambda i: (i, 0)),
        out_shape=jax.ShapeDtypeStruct((N, D), jnp.float32),
    )(parts)


def kernel(x, edge_index, edge_weights, W_f, b_f, W_b, b_b):
    src = edge_index[0]
    dst = edge_index[1]
    ar = jnp.arange(PAD, dtype=jnp.int32)
    src_p = jnp.concatenate([src, ar % N])
    dst_p = jnp.concatenate([dst, N + (ar % 16)])
    src2 = src_p.reshape(EP // K, K)
    dst2 = dst_p.reshape(EP // K, K)
    zeros = jnp.zeros((NACC, D), jnp.float32)
    parts = _sc_accumulate(x, src2, dst2, zeros)
    return _tc_normalize(parts)
